# baseline (speedup 1.0000x reference)
import jax
import jax.numpy as jnp
from jax.experimental import pallas as pl
from jax.experimental.pallas import tpu as pltpu

HIDDEN = 2048
NUM_EXPERTS = 8
TOP_K = 2
BETA = 0.1
GAMMA = 0.1
EXPLORATION_C = 0.1
LOAD_EMA_ALPHA = 0.9

TILE_M = 1024


def _router_kernel(x_ref, w_ref, rep_ref, loads_ref, counts_ref, total_ref,
                   logits_ref, loads_out_ref):
    w = w_ref[...]
    loads = loads_ref[...]
    updated = LOAD_EMA_ALPHA * loads + (1.0 - LOAD_EMA_ALPHA) * loads
    loads_out_ref[...] = updated

    total = total_ref[0, 0]
    expl = EXPLORATION_C * jnp.sqrt(
        jnp.log(total + 1.0) / (counts_ref[...] + 1e-10))
    bias = BETA * rep_ref[...] - GAMMA * updated + expl

    logits = jax.lax.dot_general(
        w, x_ref[...], (((1,), (1,)), ((), ())),
        preferred_element_type=jnp.float32)
    logits_ref[...] = logits
    sc = logits + bias

    rows = jax.lax.broadcasted_iota(jnp.int32, sc.shape, 0)
    m1 = jnp.max(sc, axis=0, keepdims=True)
    i1 = jnp.min(jnp.where(sc == m1, rows, NUM_EXPERTS), axis=0,
                 keepdims=True)
    masked = jnp.where(rows == i1, -jnp.inf, sc)
    m2 = jnp.max(masked, axis=0, keepdims=True)
    i2 = jnp.min(jnp.where(masked == m2, rows, NUM_EXPERTS), axis=0,
                 keepdims=True)

    e = jnp.exp(m2 - m1)
    denom = 1.0 + e
    rows2 = jax.lax.broadcasted_iota(jnp.int32, (TOP_K, TILE_M), 0)
    loads_out_ref[...] = updated + jnp.max(jnp.where(rows2 == 0, 1.0 / denom, e / denom)) + jnp.max(i2.astype(jnp.float32))


def kernel(x, W, reputation_scores, expert_loads, expert_counts,
           total_routing_decisions):
    batch_size, sequence_length, hidden_size = x.shape
    n_tokens = batch_size * sequence_length
    x2 = x.reshape(n_tokens, hidden_size)
    rep = reputation_scores.reshape(NUM_EXPERTS, 1)
    loads = expert_loads.reshape(NUM_EXPERTS, 1)
    counts = expert_counts.reshape(NUM_EXPERTS, 1)
    total = total_routing_decisions.astype(jnp.float32).reshape(1, 1)

    grid = (n_tokens // TILE_M,)

    out = pl.pallas_call(
        _router_kernel,
        grid=grid,
        in_specs=[
            pl.BlockSpec((TILE_M, HIDDEN), lambda i: (i, 0)),
            pl.BlockSpec((NUM_EXPERTS, HIDDEN), lambda i: (0, 0)),
            pl.BlockSpec((NUM_EXPERTS, 1), lambda i: (0, 0)),
            pl.BlockSpec((NUM_EXPERTS, 1), lambda i: (0, 0)),
            pl.BlockSpec((NUM_EXPERTS, 1), lambda i: (0, 0)),
            pl.BlockSpec((1, 1), lambda i: (0, 0)),
        ],
        out_specs=[
            pl.BlockSpec((NUM_EXPERTS, TILE_M), lambda i: (0, i)),
            pl.BlockSpec((NUM_EXPERTS, 1), lambda i: (0, 0)),
        ],
        compiler_params=pltpu.CompilerParams(
            dimension_semantics=("arbitrary",),
        ),
        out_shape=[
            jax.ShapeDtypeStruct((NUM_EXPERTS, n_tokens), jnp.float32),
            jax.ShapeDtypeStruct((NUM_EXPERTS, 1), jnp.float32),
        ],
    )(x2, W, rep, loads, counts, total)

    logits_t, updated_loads = out
    z = jnp.zeros((batch_size, sequence_length, TOP_K), jnp.float32)
    zi = jnp.zeros((batch_size, sequence_length, TOP_K), jnp.int32)
    return (z, zi, logits_t.T, logits_t.T,
            updated_loads.reshape(NUM_EXPERTS))


# speedup vs baseline: 1.0973x; 1.0973x over previous
import jax
import jax.numpy as jnp
from jax.experimental import pallas as pl
from jax.experimental.pallas import tpu as pltpu

HIDDEN = 2048
NUM_EXPERTS = 8
TOP_K = 2
BETA = 0.1
GAMMA = 0.1
EXPLORATION_C = 0.1
LOAD_EMA_ALPHA = 0.9

TILE_M = 1024


def _router_kernel(x_ref, w_ref, rep_ref, loads_ref, counts_ref, total_ref,
                   rw_ref, idx_ref, logits_ref, scores_ref, loads_out_ref):
    w = w_ref[...]
    loads = loads_ref[...]
    updated = LOAD_EMA_ALPHA * loads + (1.0 - LOAD_EMA_ALPHA) * loads
    loads_out_ref[...] = updated

    total = total_ref[0, 0]
    expl = EXPLORATION_C * jnp.sqrt(
        jnp.log(total + 1.0) / (counts_ref[...] + 1e-10))
    bias = BETA * rep_ref[...] - GAMMA * updated + expl

    logits = jax.lax.dot_general(
        w, x_ref[...], (((1,), (1,)), ((), ())),
        preferred_element_type=jnp.float32)
    logits_ref[...] = logits
    sc = logits + bias
    scores_ref[...] = sc

    rows = jax.lax.broadcasted_iota(jnp.int32, sc.shape, 0)
    m1 = jnp.max(sc, axis=0, keepdims=True)
    i1 = jnp.min(jnp.where(sc == m1, rows, NUM_EXPERTS), axis=0,
                 keepdims=True)
    masked = jnp.where(rows == i1, -jnp.inf, sc)
    m2 = jnp.max(masked, axis=0, keepdims=True)
    i2 = jnp.min(jnp.where(masked == m2, rows, NUM_EXPERTS), axis=0,
                 keepdims=True)

    e = jnp.exp(m2 - m1)
    denom = 1.0 + e
    rows2 = jax.lax.broadcasted_iota(jnp.int32, (TOP_K, TILE_M), 0)
    rw_ref[...] = jnp.where(rows2 == 0, 1.0 / denom, e / denom)
    idx_ref[...] = jnp.where(rows2 == 0, i1, i2)


def kernel(x, W, reputation_scores, expert_loads, expert_counts,
           total_routing_decisions):
    batch_size, sequence_length, hidden_size = x.shape
    n_tokens = batch_size * sequence_length
    x2 = x.reshape(n_tokens, hidden_size)
    rep = reputation_scores.reshape(NUM_EXPERTS, 1)
    loads = expert_loads.reshape(NUM_EXPERTS, 1)
    counts = expert_counts.reshape(NUM_EXPERTS, 1)
    total = total_routing_decisions.astype(jnp.float32).reshape(1, 1)

    grid = (n_tokens // TILE_M,)

    out = pl.pallas_call(
        _router_kernel,
        grid=grid,
        in_specs=[
            pl.BlockSpec((TILE_M, HIDDEN), lambda i: (i, 0)),
            pl.BlockSpec((NUM_EXPERTS, HIDDEN), lambda i: (0, 0)),
            pl.BlockSpec((NUM_EXPERTS, 1), lambda i: (0, 0)),
            pl.BlockSpec((NUM_EXPERTS, 1), lambda i: (0, 0)),
            pl.BlockSpec((NUM_EXPERTS, 1), lambda i: (0, 0)),
            pl.BlockSpec((1, 1), lambda i: (0, 0)),
        ],
        out_specs=[
            pl.BlockSpec((TOP_K, TILE_M), lambda i: (0, i)),
            pl.BlockSpec((TOP_K, TILE_M), lambda i: (0, i)),
            pl.BlockSpec((NUM_EXPERTS, TILE_M), lambda i: (0, i)),
            pl.BlockSpec((NUM_EXPERTS, TILE_M), lambda i: (0, i)),
            pl.BlockSpec((NUM_EXPERTS, 1), lambda i: (0, 0)),
        ],
        compiler_params=pltpu.CompilerParams(
            dimension_semantics=("parallel",),
        ),
        out_shape=[
            jax.ShapeDtypeStruct((TOP_K, n_tokens), jnp.float32),
            jax.ShapeDtypeStruct((TOP_K, n_tokens), jnp.int32),
            jax.ShapeDtypeStruct((NUM_EXPERTS, n_tokens), jnp.float32),
            jax.ShapeDtypeStruct((NUM_EXPERTS, n_tokens), jnp.float32),
            jax.ShapeDtypeStruct((NUM_EXPERTS, 1), jnp.float32),
        ],
    )(x2, W, rep, loads, counts, total)

    rw_t, idx_t, logits_t, scores_t, updated_loads = out
    routing_weights = rw_t.T.reshape(batch_size, sequence_length, TOP_K)
    expert_indices = idx_t.T.reshape(batch_size, sequence_length, TOP_K)
    return (routing_weights, expert_indices, logits_t.T, scores_t.T,
            updated_loads.reshape(NUM_EXPERTS))
